# baseline (speedup 1.0000x reference)
import functools

import jax
import jax.numpy as jnp
from jax import lax
from jax.experimental import pallas as pl
from jax.experimental.pallas import tpu as pltpu
from jax.experimental.pallas import tpu_sc as plsc

_NN = 100000
_NE = 1600000
_NG = 256
_DN = 128
_DE = 32
_DO = 64

_NW = 32
_NP = 4
_EP = _NE // _NP
_C = 640
_CR = _C // 4
_NCH = _EP // _C
_TRIP2 = 10
_GRP = _C // 16

_Q = _NE // 4

_sc_mesh = plsc.VectorSubcoreMesh(core_axis_name="c", subcore_axis_name="s")


@functools.partial(
    pl.kernel,
    mesh=_sc_mesh,
    compiler_params=pltpu.CompilerParams(needs_layout_passes=False),
    out_type=jax.ShapeDtypeStruct((_NW, _NG * _DE), jnp.float32),
    scratch_types=[
        pltpu.VMEM((_NN // 4,), jnp.int32),
        pltpu.VMEM((_C,), jnp.int32),
        pltpu.VMEM((_C,), jnp.int32),
        pltpu.VMEM((_CR, 128), jnp.float32),
        pltpu.VMEM((_CR, 128), jnp.float32),
        pltpu.VMEM((_NG * _DE,), jnp.float32),
        pltpu.SemaphoreType.DMA,
        pltpu.SemaphoreType.DMA,
    ],
)
def _edge_agg(col_hbm, ea_hbm, batch_hbm, out_hbm, batch_v, col_b0, col_b1,
              ea_b0, ea_b1, acc_v, sem0, sem1):
    wid = lax.axis_index("s") * 2 + lax.axis_index("c")
    pltpu.sync_copy(batch_hbm, batch_v)

    zeros = jnp.zeros((16,), jnp.float32)

    def zero_body(z, carry):
        acc_v[pl.ds(z * 16, 16)] = zeros
        return carry

    lax.fori_loop(0, _NG * _DE // 16, zero_body, 0)

    iota = lax.iota(jnp.int32, 16)

    def _descs(local, colb, eab, sem):
        k = local * _NW + wid
        dc = pltpu.make_async_copy(col_hbm.at[pl.ds(k * _C, _C)], colb, sem)
        de = pltpu.make_async_copy(ea_hbm.at[pl.ds(k * _CR, _CR), :], eab,
                                   sem)
        return k, dc, de

    def start(local, colb, eab, sem):
        k, dc, de = _descs(local, colb, eab, sem)

        @pl.when(k < _NCH)
        def _():
            dc.start()
            de.start()

    def process(local, colb, eab, sem, nxt):
        k, dc, de = _descs(local, colb, eab, sem)

        @pl.when(k < _NCH)
        def _():
            dc.wait()
            de.wait()
            start(*nxt)

            def grp_body(g, c2):
                cv = colb[pl.ds(g * 16, 16)]
                word = plsc.load_gather(batch_v, [cv >> 2])
                gv = ((word >> ((cv & 3) << 3)) & 255) * _DE
                for j in range(16):
                    row = gv[j]
                    r = g * 4 + j // 4
                    c0 = (j % 4) * 32
                    v0 = eab[r, pl.ds(c0, 16)]
                    v1 = eab[r, pl.ds(c0 + 16, 16)]
                    plsc.addupdate_scatter(acc_v, [row + iota], v0)
                    plsc.addupdate_scatter(acc_v, [row + (iota + 16)], v1)
                return c2

            lax.fori_loop(0, _GRP, grp_body, 0)

    start(0, col_b0, ea_b0, sem0)

    def body2(i2, carry):
        l0 = i2 * 2
        process(l0, col_b0, ea_b0, sem0, (l0 + 1, col_b1, ea_b1, sem1))
        process(l0 + 1, col_b1, ea_b1, sem1, (l0 + 2, col_b0, ea_b0, sem0))
        return carry

    lax.fori_loop(0, _TRIP2, body2, 0)
    pltpu.sync_copy(acc_v, out_hbm.at[wid])


_R = 2000
_NS = _NN // _R


def _node_body(b_ref, x_ref, o_ref, acc_ref):
    s = pl.program_id(0)

    @pl.when(s == 0)
    def _():
        acc_ref[...] = jnp.zeros_like(acc_ref)

    bt = b_ref[0, 0, :]
    onehot = (lax.broadcasted_iota(jnp.int32, (_NG, _R), 0)
              == bt[None, :]).astype(jnp.float32)
    acc_ref[...] += jnp.dot(onehot, x_ref[...],
                            preferred_element_type=jnp.float32)

    @pl.when(s == _NS - 1)
    def _():
        o_ref[...] = acc_ref[...]


_node_call = pl.pallas_call(
    _node_body,
    grid=(_NS,),
    in_specs=[
        pl.BlockSpec((1, 1, _R), lambda i: (i, 0, 0)),
        pl.BlockSpec((_R, _DN), lambda i: (i, 0)),
    ],
    out_specs=pl.BlockSpec((_NG, _DN), lambda i: (0, 0)),
    out_shape=jax.ShapeDtypeStruct((_NG, _DN), jnp.float32),
    scratch_shapes=[pltpu.VMEM((_NG, _DN), jnp.float32)],
)


def _comb_body(nag_ref, ep_ref, w_ref, b_ref, o_ref):
    eag = jnp.sum(ep_ref[...], axis=0)
    out = jnp.dot(nag_ref[...], w_ref[0:_DN, :],
                  preferred_element_type=jnp.float32)
    out = out + jnp.dot(eag, w_ref[_DN:_DN + _DE, :],
                        preferred_element_type=jnp.float32)
    o_ref[...] = out + b_ref[...]


_comb_call = pl.pallas_call(
    _comb_body,
    out_shape=jax.ShapeDtypeStruct((_NG, _DO), jnp.float32),
)


@jax.jit
def kernel(x, edge_index, edge_attr, u, batch, W, b):
    col = edge_index[1]
    b4 = batch.reshape(_NN // 4, 4)
    batch_p = (b4[:, 0] | (b4[:, 1] << 8) | (b4[:, 2] << 16)
               | (b4[:, 3] << 24))
    eps = []
    for p in range(_NP):
        ea_p = edge_attr[p * _EP:(p + 1) * _EP].reshape(_EP // 4, 128)
        col_p = col[p * _EP:(p + 1) * _EP]
        eps.append(_edge_agg(col_p, ea_p, batch_p))
    ep = jnp.concatenate(eps, axis=0)
    nag = _node_call(batch.reshape(_NS, 1, _R), x)
    return _comb_call(nag, ep.reshape(_NP * _NW, _NG, _DE), W,
                      b.reshape(1, _DO))


# speedup vs baseline: 1.4225x; 1.4225x over previous
import functools

import jax
import jax.numpy as jnp
from jax import lax
from jax.experimental import pallas as pl
from jax.experimental.pallas import tpu as pltpu
from jax.experimental.pallas import tpu_sc as plsc

_NN = 100000
_NE = 1600000
_NG = 256
_DN = 128
_DE = 32
_DO = 64

_NW = 32
_C = 1280
_CR = _C // 4
_NCH = _NE // _C
_TRIP2 = 20
_GRP = _C // 16

_Q = _NE // 4

_sc_mesh = plsc.VectorSubcoreMesh(core_axis_name="c", subcore_axis_name="s")


@functools.partial(
    pl.kernel,
    mesh=_sc_mesh,
    compiler_params=pltpu.CompilerParams(needs_layout_passes=False),
    out_type=jax.ShapeDtypeStruct((_NW, _NG * _DE), jnp.float32),
    scratch_types=[
        pltpu.VMEM((_NN // 4,), jnp.int32),
        pltpu.VMEM((_C,), jnp.int32),
        pltpu.VMEM((_C,), jnp.int32),
        pltpu.VMEM((_CR, 128), jnp.float32),
        pltpu.VMEM((_CR, 128), jnp.float32),
        pltpu.VMEM((_NG * _DE,), jnp.float32),
        pltpu.SemaphoreType.DMA,
        pltpu.SemaphoreType.DMA,
    ],
)
def _edge_agg(col_hbm, ea_hbm, batch_hbm, out_hbm, batch_v, col_b0, col_b1,
              ea_b0, ea_b1, acc_v, sem0, sem1):
    wid = lax.axis_index("s") * 2 + lax.axis_index("c")
    pltpu.sync_copy(batch_hbm, batch_v)

    zeros = jnp.zeros((16,), jnp.float32)

    def zero_body(z, carry):
        acc_v[pl.ds(z * 16, 16)] = zeros
        return carry

    lax.fori_loop(0, _NG * _DE // 16, zero_body, 0)

    iota = lax.iota(jnp.int32, 16)

    def _descs(local, colb, eab, sem):
        k = local * _NW + wid
        dc = pltpu.make_async_copy(col_hbm.at[pl.ds(k * _C, _C)], colb, sem)
        de = pltpu.make_async_copy(ea_hbm.at[pl.ds(k * _CR, _CR), :], eab,
                                   sem)
        return k, dc, de

    def start(local, colb, eab, sem):
        k, dc, de = _descs(local, colb, eab, sem)

        @pl.when(k < _NCH)
        def _():
            dc.start()
            de.start()

    def process(local, colb, eab, sem, nxt):
        k, dc, de = _descs(local, colb, eab, sem)

        @pl.when(k < _NCH)
        def _():
            dc.wait()
            de.wait()
            start(*nxt)

            def grp_body(g, c2):
                cv = colb[pl.ds(g * 16, 16)]
                word = plsc.load_gather(batch_v, [cv >> 2])
                gv = ((word >> ((cv & 3) << 3)) & 255) * _DE
                for j in range(16):
                    row = gv[j]
                    r = g * 4 + j // 4
                    c0 = (j % 4) * 32
                    v0 = eab[r, pl.ds(c0, 16)]
                    v1 = eab[r, pl.ds(c0 + 16, 16)]
                    plsc.addupdate_scatter(acc_v, [row + iota], v0)
                    plsc.addupdate_scatter(acc_v, [row + (iota + 16)], v1)
                return c2

            lax.fori_loop(0, _GRP, grp_body, 0)

    start(0, col_b0, ea_b0, sem0)

    def body2(i2, carry):
        l0 = i2 * 2
        process(l0, col_b0, ea_b0, sem0, (l0 + 1, col_b1, ea_b1, sem1))
        process(l0 + 1, col_b1, ea_b1, sem1, (l0 + 2, col_b0, ea_b0, sem0))
        return carry

    lax.fori_loop(0, _TRIP2, body2, 0)
    pltpu.sync_copy(acc_v, out_hbm.at[wid])


_R = 2000
_NS = _NN // _R


def _node_body(b_ref, x_ref, o_ref, acc_ref):
    s = pl.program_id(0)

    @pl.when(s == 0)
    def _():
        acc_ref[...] = jnp.zeros_like(acc_ref)

    bt = b_ref[0, 0, :]
    onehot = (lax.broadcasted_iota(jnp.int32, (_NG, _R), 0)
              == bt[None, :]).astype(jnp.float32)
    acc_ref[...] += jnp.dot(onehot, x_ref[...],
                            preferred_element_type=jnp.float32)

    @pl.when(s == _NS - 1)
    def _():
        o_ref[...] = acc_ref[...]


_node_call = pl.pallas_call(
    _node_body,
    grid=(_NS,),
    in_specs=[
        pl.BlockSpec((1, 1, _R), lambda i: (i, 0, 0)),
        pl.BlockSpec((_R, _DN), lambda i: (i, 0)),
    ],
    out_specs=pl.BlockSpec((_NG, _DN), lambda i: (0, 0)),
    out_shape=jax.ShapeDtypeStruct((_NG, _DN), jnp.float32),
    scratch_shapes=[pltpu.VMEM((_NG, _DN), jnp.float32)],
)


def _comb_body(nag_ref, ep_ref, w_ref, b_ref, o_ref):
    eag = jnp.sum(ep_ref[...], axis=0)
    out = jnp.dot(nag_ref[...], w_ref[0:_DN, :],
                  preferred_element_type=jnp.float32)
    out = out + jnp.dot(eag, w_ref[_DN:_DN + _DE, :],
                        preferred_element_type=jnp.float32)
    o_ref[...] = out + b_ref[...]


_comb_call = pl.pallas_call(
    _comb_body,
    out_shape=jax.ShapeDtypeStruct((_NG, _DO), jnp.float32),
)


@jax.jit
def kernel(x, edge_index, edge_attr, u, batch, W, b):
    col = edge_index[1]
    b4 = batch.reshape(_NN // 4, 4)
    batch_p = (b4[:, 0] | (b4[:, 1] << 8) | (b4[:, 2] << 16)
               | (b4[:, 3] << 24))
    ea2 = edge_attr.reshape(_NE // 4, 128)
    ep = _edge_agg(col, ea2, batch_p)
    nag = _node_call(batch.reshape(_NS, 1, _R), x)
    return _comb_call(nag, ep.reshape(_NW, _NG, _DE), W,
                      b.reshape(1, _DO))


# speedup vs baseline: 1.4286x; 1.0043x over previous
import functools

import jax
import jax.numpy as jnp
from jax import lax
from jax.experimental import pallas as pl
from jax.experimental.pallas import tpu as pltpu
from jax.experimental.pallas import tpu_sc as plsc

_NN = 100000
_NE = 1600000
_NG = 256
_DN = 128
_DE = 32
_DO = 64

_NW = 32
_C = 640
_CR = _C // 4
_NCH = _NE // _C
_TRIP2 = 40
_GRP = _C // 16

_sc_mesh = plsc.VectorSubcoreMesh(core_axis_name="c", subcore_axis_name="s")


@functools.partial(
    pl.kernel,
    mesh=_sc_mesh,
    compiler_params=pltpu.CompilerParams(needs_layout_passes=False),
    out_type=jax.ShapeDtypeStruct((_NW, _NG * _DE), jnp.float32),
    scratch_types=[
        pltpu.VMEM((_NN // 4,), jnp.int32),
        pltpu.VMEM((_C,), jnp.int32),
        pltpu.VMEM((_C,), jnp.int32),
        pltpu.VMEM((_CR, 128), jnp.float32),
        pltpu.VMEM((_CR, 128), jnp.float32),
        pltpu.VMEM((_NG * _DE,), jnp.float32),
        pltpu.SemaphoreType.DMA,
        pltpu.SemaphoreType.DMA,
    ],
)
def _edge_agg(col_hbm, ea_hbm, batch_hbm, out_hbm, batch_v, col_b0, col_b1,
              ea_b0, ea_b1, acc_v, sem0, sem1):
    wid = lax.axis_index("s") * 2 + lax.axis_index("c")
    pltpu.sync_copy(batch_hbm, batch_v)

    zeros = jnp.zeros((16,), jnp.float32)

    def zero_body(z, carry):
        acc_v[pl.ds(z * 16, 16)] = zeros
        return carry

    lax.fori_loop(0, _NG * _DE // 16, zero_body, 0)

    iota = lax.iota(jnp.int32, 16)

    def _descs(local, colb, eab, sem):
        k = local * _NW + wid
        dc = pltpu.make_async_copy(col_hbm.at[pl.ds(k * _C, _C)], colb, sem)
        de = pltpu.make_async_copy(ea_hbm.at[pl.ds(k * _CR, _CR), :], eab,
                                   sem)
        return k, dc, de

    def start(local, colb, eab, sem):
        k, dc, de = _descs(local, colb, eab, sem)

        @pl.when(k < _NCH)
        def _():
            dc.start()
            de.start()

    def process(local, colb, eab, sem, nxt):
        k, dc, de = _descs(local, colb, eab, sem)

        @pl.when(k < _NCH)
        def _():
            dc.wait()
            de.wait()
            start(*nxt)

            def grp_body(g, c2):
                cv = colb[pl.ds(g * 16, 16)]
                word = plsc.load_gather(batch_v, [cv >> 2])
                gv = ((word >> ((cv & 3) << 3)) & 255) * _DE
                for j in range(16):
                    row = gv[j]
                    r = g * 4 + j // 4
                    c0 = (j % 4) * 32
                    v0 = eab[r, pl.ds(c0, 16)]
                    v1 = eab[r, pl.ds(c0 + 16, 16)]
                    plsc.addupdate_scatter(acc_v, [row + iota], v0)
                    plsc.addupdate_scatter(acc_v, [row + (iota + 16)], v1)
                return c2

            lax.fori_loop(0, _GRP, grp_body, 0)

    start(0, col_b0, ea_b0, sem0)

    def body2(i2, carry):
        l0 = i2 * 2
        process(l0, col_b0, ea_b0, sem0, (l0 + 1, col_b1, ea_b1, sem1))
        process(l0 + 1, col_b1, ea_b1, sem1, (l0 + 2, col_b0, ea_b0, sem0))
        return carry

    lax.fori_loop(0, _TRIP2, body2, 0)
    pltpu.sync_copy(acc_v, out_hbm.at[wid])


_R = 2000
_NS = _NN // _R


def _node_body(b_ref, x_ref, o_ref, acc_ref):
    s = pl.program_id(0)

    @pl.when(s == 0)
    def _():
        acc_ref[...] = jnp.zeros_like(acc_ref)

    bt = b_ref[0, 0, :]
    onehot = (lax.broadcasted_iota(jnp.int32, (_NG, _R), 0)
              == bt[None, :]).astype(jnp.float32)
    acc_ref[...] += jnp.dot(onehot, x_ref[...],
                            preferred_element_type=jnp.float32)

    @pl.when(s == _NS - 1)
    def _():
        o_ref[...] = acc_ref[...]


_node_call = pl.pallas_call(
    _node_body,
    grid=(_NS,),
    in_specs=[
        pl.BlockSpec((1, 1, _R), lambda i: (i, 0, 0)),
        pl.BlockSpec((_R, _DN), lambda i: (i, 0)),
    ],
    out_specs=pl.BlockSpec((_NG, _DN), lambda i: (0, 0)),
    out_shape=jax.ShapeDtypeStruct((_NG, _DN), jnp.float32),
    scratch_shapes=[pltpu.VMEM((_NG, _DN), jnp.float32)],
)


def _comb_body(nag_ref, ep_ref, w_ref, b_ref, o_ref):
    eag = jnp.sum(ep_ref[...], axis=0)
    out = jnp.dot(nag_ref[...], w_ref[0:_DN, :],
                  preferred_element_type=jnp.float32)
    out = out + jnp.dot(eag, w_ref[_DN:_DN + _DE, :],
                        preferred_element_type=jnp.float32)
    o_ref[...] = out + b_ref[...]


_comb_call = pl.pallas_call(
    _comb_body,
    out_shape=jax.ShapeDtypeStruct((_NG, _DO), jnp.float32),
)


@jax.jit
def kernel(x, edge_index, edge_attr, u, batch, W, b):
    col = edge_index[1]
    b4 = batch.reshape(_NN // 4, 4)
    batch_p = (b4[:, 0] | (b4[:, 1] << 8) | (b4[:, 2] << 16)
               | (b4[:, 3] << 24))
    ea2 = edge_attr.reshape(_NE // 4, 128)
    ep = _edge_agg(col, ea2, batch_p)
    nag = _node_call(batch.reshape(_NS, 1, _R), x)
    return _comb_call(nag, ep.reshape(_NW, _NG, _DE), W,
                      b.reshape(1, _DO))
